# Initial kernel scaffold; baseline (speedup 1.0000x reference)
import jax
import jax.numpy as jnp
from jax.experimental import pallas as pl


def kernel(x, lengths, embed_table, W_ih, W_hh, b_ih, b_hh):
    raise NotImplementedError("write your pallas kernel here")


# speedup vs baseline: 2.1887x; 2.1887x over previous
import functools

import jax
import jax.numpy as jnp
from jax import lax
from jax.experimental import pallas as pl
from jax.experimental.pallas import tpu as pltpu
from jax.experimental.pallas import tpu_sc as plsc

VOCAB = 100000
WORD_DIM = 128
EMBED_SIZE = 512
BATCH = 1024
SEQ = 20


_N_ROWS = BATCH * SEQ
_IDX_CHUNK = 128


def _sc_gather(idx_flat, table):
    info = plsc.get_sparse_core_info()
    nc, ns = info.num_cores, info.num_subcores
    nw = nc * ns
    rows_per_w = _N_ROWS // nw
    assert rows_per_w % _IDX_CHUNK == 0
    n_chunks = rows_per_w // _IDX_CHUNK

    mesh = plsc.VectorSubcoreMesh(core_axis_name="c", subcore_axis_name="s")

    @functools.partial(
        pl.kernel,
        mesh=mesh,
        out_type=jax.ShapeDtypeStruct((_N_ROWS, WORD_DIM), jnp.float32),
        scratch_types=[
            pltpu.VMEM((rows_per_w,), jnp.int32),
            pltpu.VMEM((rows_per_w, WORD_DIM), jnp.float32),
            pltpu.SemaphoreType.DMA,
        ],
    )
    def gather_k(idx_hbm, table_hbm, out_hbm, idx_v, rows_v, sem):
        wid = lax.axis_index("s") * nc + lax.axis_index("c")
        base = wid * rows_per_w
        pltpu.sync_copy(idx_hbm.at[pl.ds(base, rows_per_w)], idx_v)
        copies = []
        for j in range(n_chunks):
            sl = pl.ds(j * _IDX_CHUNK, _IDX_CHUNK)
            copies.append(
                pltpu.async_copy(table_hbm.at[idx_v.at[sl]], rows_v.at[sl], sem)
            )
        for c in copies:
            c.wait()
        pltpu.sync_copy(rows_v, out_hbm.at[pl.ds(base, rows_per_w)])

    return gather_k(idx_flat, table)


_BB = 256


def _gru_block(cap_ref, len_ref, wih_ref, whh_ref, bih_ref, bhh_ref, out_ref):
    cap = cap_ref[...]
    wih = wih_ref[...]
    whh = whh_ref[...]
    bih = bih_ref[...]
    bhh = bhh_ref[...]
    lens = len_ref[...]

    h = jnp.zeros((_BB, EMBED_SIZE), dtype=jnp.float32)
    acc = jnp.full((_BB, EMBED_SIZE), jnp.finfo(jnp.float32).min, jnp.float32)

    dn = (((1,), (1,)), ((), ()))
    H = EMBED_SIZE
    for t in range(SEQ):
        xt = cap[:, t, :]
        gi = lax.dot_general(xt, wih, dn, preferred_element_type=jnp.float32)
        gi = gi + bih
        gh = lax.dot_general(h, whh, dn, preferred_element_type=jnp.float32)
        gh = gh + bhh
        r = jax.nn.sigmoid(gi[:, :H] + gh[:, :H])
        z = jax.nn.sigmoid(gi[:, H:2 * H] + gh[:, H:2 * H])
        n = jnp.tanh(gi[:, 2 * H:] + r * gh[:, 2 * H:])
        h = (1.0 - z) * n + z * h
        valid = t < lens
        acc = jnp.where(valid, jnp.maximum(acc, h), acc)

    out_ref[...] = acc


def _tc_gru(cap_emb, lengths2d, W_ih, W_hh, b_ih2d, b_hh2d):
    grid = BATCH // _BB
    return pl.pallas_call(
        _gru_block,
        grid=(grid,),
        in_specs=[
            pl.BlockSpec((_BB, SEQ, WORD_DIM), lambda i: (i, 0, 0)),
            pl.BlockSpec((_BB, 1), lambda i: (i, 0)),
            pl.BlockSpec((3 * EMBED_SIZE, WORD_DIM), lambda i: (0, 0)),
            pl.BlockSpec((3 * EMBED_SIZE, EMBED_SIZE), lambda i: (0, 0)),
            pl.BlockSpec((1, 3 * EMBED_SIZE), lambda i: (0, 0)),
            pl.BlockSpec((1, 3 * EMBED_SIZE), lambda i: (0, 0)),
        ],
        out_specs=pl.BlockSpec((_BB, EMBED_SIZE), lambda i: (i, 0)),
        out_shape=jax.ShapeDtypeStruct((BATCH, EMBED_SIZE), jnp.float32),
    )(cap_emb, lengths2d, W_ih, W_hh, b_ih2d, b_hh2d)


def kernel(x, lengths, embed_table, W_ih, W_hh, b_ih, b_hh):
    idx_flat = x.reshape(-1).astype(jnp.int32)
    cap_flat = _sc_gather(idx_flat, embed_table)
    cap_emb = cap_flat.reshape(BATCH, SEQ, WORD_DIM)
    out = _tc_gru(
        cap_emb,
        lengths.reshape(BATCH, 1).astype(jnp.int32),
        W_ih,
        W_hh,
        b_ih.reshape(1, -1),
        b_hh.reshape(1, -1),
    )
    return (out, cap_emb)


# speedup vs baseline: 2.2339x; 1.0207x over previous
import functools

import jax
import jax.numpy as jnp
from jax import lax
from jax.experimental import pallas as pl
from jax.experimental.pallas import tpu as pltpu
from jax.experimental.pallas import tpu_sc as plsc

VOCAB = 100000
WORD_DIM = 128
EMBED_SIZE = 512
BATCH = 1024
SEQ = 20


_N_ROWS = BATCH * SEQ
_IDX_CHUNK = 128


def _sc_gather(idx_flat, table):
    info = plsc.get_sparse_core_info()
    nc, ns = info.num_cores, info.num_subcores
    nw = nc * ns
    rows_per_w = _N_ROWS // nw
    assert rows_per_w % _IDX_CHUNK == 0
    n_chunks = rows_per_w // _IDX_CHUNK

    mesh = plsc.VectorSubcoreMesh(core_axis_name="c", subcore_axis_name="s")

    @functools.partial(
        pl.kernel,
        mesh=mesh,
        out_type=jax.ShapeDtypeStruct((_N_ROWS, WORD_DIM), jnp.float32),
        scratch_types=[
            pltpu.VMEM((rows_per_w,), jnp.int32),
            pltpu.VMEM((rows_per_w, WORD_DIM), jnp.float32),
            pltpu.SemaphoreType.DMA,
        ],
    )
    def gather_k(idx_hbm, table_hbm, out_hbm, idx_v, rows_v, sem):
        wid = lax.axis_index("s") * nc + lax.axis_index("c")
        base = wid * rows_per_w
        pltpu.sync_copy(idx_hbm.at[pl.ds(base, rows_per_w)], idx_v)
        copies = []
        for j in range(n_chunks):
            sl = pl.ds(j * _IDX_CHUNK, _IDX_CHUNK)
            copies.append(
                pltpu.async_copy(table_hbm.at[idx_v.at[sl]], rows_v.at[sl], sem)
            )
        for c in copies:
            c.wait()
        pltpu.sync_copy(rows_v, out_hbm.at[pl.ds(base, rows_per_w)])

    return gather_k(idx_flat, table)


_BB = 256


def _gru_block(cap_ref, len_ref, wih_ref, whh_ref, bih_ref, bhh_ref, out_ref):
    cap = cap_ref[...]
    wih = wih_ref[...]
    whh = whh_ref[...]
    bih = bih_ref[...]
    bhh = bhh_ref[...]
    lens = len_ref[...]

    h = jnp.zeros((_BB, EMBED_SIZE), dtype=jnp.float32)
    acc = jnp.full((_BB, EMBED_SIZE), jnp.finfo(jnp.float32).min, jnp.float32)

    dn = (((1,), (1,)), ((), ()))
    H = EMBED_SIZE
    wih_b = wih.astype(jnp.bfloat16)
    whh_b = whh.astype(jnp.bfloat16)
    for t in range(SEQ):
        xt = cap[:, t, :].astype(jnp.bfloat16)
        gi = lax.dot_general(xt, wih_b, dn, preferred_element_type=jnp.float32)
        gi = gi + bih
        gh = lax.dot_general(h.astype(jnp.bfloat16), whh_b, dn,
                             preferred_element_type=jnp.float32)
        gh = gh + bhh
        r = jax.nn.sigmoid(gi[:, :H] + gh[:, :H])
        z = jax.nn.sigmoid(gi[:, H:2 * H] + gh[:, H:2 * H])
        n = jnp.tanh(gi[:, 2 * H:] + r * gh[:, 2 * H:])
        h = (1.0 - z) * n + z * h
        valid = t < lens
        acc = jnp.where(valid, jnp.maximum(acc, h), acc)

    out_ref[...] = acc


def _tc_gru(cap_emb, lengths2d, W_ih, W_hh, b_ih2d, b_hh2d):
    grid = BATCH // _BB
    return pl.pallas_call(
        _gru_block,
        grid=(grid,),
        in_specs=[
            pl.BlockSpec((_BB, SEQ, WORD_DIM), lambda i: (i, 0, 0)),
            pl.BlockSpec((_BB, 1), lambda i: (i, 0)),
            pl.BlockSpec((3 * EMBED_SIZE, WORD_DIM), lambda i: (0, 0)),
            pl.BlockSpec((3 * EMBED_SIZE, EMBED_SIZE), lambda i: (0, 0)),
            pl.BlockSpec((1, 3 * EMBED_SIZE), lambda i: (0, 0)),
            pl.BlockSpec((1, 3 * EMBED_SIZE), lambda i: (0, 0)),
        ],
        out_specs=pl.BlockSpec((_BB, EMBED_SIZE), lambda i: (i, 0)),
        out_shape=jax.ShapeDtypeStruct((BATCH, EMBED_SIZE), jnp.float32),
    )(cap_emb, lengths2d, W_ih, W_hh, b_ih2d, b_hh2d)


def kernel(x, lengths, embed_table, W_ih, W_hh, b_ih, b_hh):
    idx_flat = x.reshape(-1).astype(jnp.int32)
    cap_flat = _sc_gather(idx_flat, embed_table)
    cap_emb = cap_flat.reshape(BATCH, SEQ, WORD_DIM)
    out = _tc_gru(
        cap_emb,
        lengths.reshape(BATCH, 1).astype(jnp.int32),
        W_ih,
        W_hh,
        b_ih.reshape(1, -1),
        b_hh.reshape(1, -1),
    )
    return (out, cap_emb)


# speedup vs baseline: 2.3573x; 1.0552x over previous
import functools

import jax
import jax.numpy as jnp
from jax import lax
from jax.experimental import pallas as pl
from jax.experimental.pallas import tpu as pltpu
from jax.experimental.pallas import tpu_sc as plsc

VOCAB = 100000
WORD_DIM = 128
EMBED_SIZE = 512
BATCH = 1024
SEQ = 20


_N_ROWS = BATCH * SEQ
_IDX_CHUNK = 128


def _sc_gather(idx_flat, table):
    info = plsc.get_sparse_core_info()
    nc, ns = info.num_cores, info.num_subcores
    nw = nc * ns
    rows_per_w = _N_ROWS // nw
    assert rows_per_w % _IDX_CHUNK == 0
    n_chunks = rows_per_w // _IDX_CHUNK

    mesh = plsc.VectorSubcoreMesh(core_axis_name="c", subcore_axis_name="s")

    @functools.partial(
        pl.kernel,
        mesh=mesh,
        out_type=jax.ShapeDtypeStruct((_N_ROWS, WORD_DIM), jnp.float32),
        scratch_types=[
            pltpu.VMEM((rows_per_w,), jnp.int32),
            pltpu.VMEM((rows_per_w, WORD_DIM), jnp.float32),
            pltpu.SemaphoreType.DMA,
        ],
    )
    def gather_k(idx_hbm, table_hbm, out_hbm, idx_v, rows_v, sem):
        wid = lax.axis_index("s") * nc + lax.axis_index("c")
        base = wid * rows_per_w
        pltpu.sync_copy(idx_hbm.at[pl.ds(base, rows_per_w)], idx_v)
        copies = []
        for j in range(n_chunks):
            sl = pl.ds(j * _IDX_CHUNK, _IDX_CHUNK)
            copies.append(
                pltpu.async_copy(table_hbm.at[idx_v.at[sl]], rows_v.at[sl], sem)
            )
        for c in copies:
            c.wait()
        pltpu.sync_copy(rows_v, out_hbm.at[pl.ds(base, rows_per_w)])

    return gather_k(idx_flat, table)


_BB = 256


def _gru_block(cap_ref, len_ref, wih_ref, whh_ref, bih_ref, bhh_ref,
               out_ref, cap3_ref):
    wih_b = wih_ref[...].astype(jnp.bfloat16)
    whh_b = whh_ref[...].astype(jnp.bfloat16)
    bih = bih_ref[...]
    bhh = bhh_ref[...]
    lens = len_ref[...]

    h = jnp.zeros((_BB, EMBED_SIZE), dtype=jnp.float32)
    acc = jnp.full((_BB, EMBED_SIZE), jnp.finfo(jnp.float32).min, jnp.float32)

    dn = (((1,), (1,)), ((), ()))
    H = EMBED_SIZE
    bsum = bih + bhh
    for t in range(SEQ):
        xt = cap_ref[t]
        cap3_ref[:, t, :] = xt
        gi = lax.dot_general(xt.astype(jnp.bfloat16), wih_b, dn,
                             preferred_element_type=jnp.float32)
        gh = lax.dot_general(h.astype(jnp.bfloat16), whh_b, dn,
                             preferred_element_type=jnp.float32)
        rz = gi[:, :2 * H] + gh[:, :2 * H] + bsum[:, :2 * H]
        rz = 0.5 * jnp.tanh(0.5 * rz) + 0.5
        r = rz[:, :H]
        z = rz[:, H:]
        hn = gh[:, 2 * H:] + bhh[:, 2 * H:]
        n = jnp.tanh(gi[:, 2 * H:] + bih[:, 2 * H:] + r * hn)
        h = n + z * (h - n)
        valid = t < lens
        acc = jnp.where(valid, jnp.maximum(acc, h), acc)

    out_ref[...] = acc


def _tc_gru(cap_tm, lengths2d, W_ih, W_hh, b_ih2d, b_hh2d):
    grid = BATCH // _BB
    return pl.pallas_call(
        _gru_block,
        grid=(grid,),
        in_specs=[
            pl.BlockSpec((SEQ, _BB, WORD_DIM), lambda i: (0, i, 0)),
            pl.BlockSpec((_BB, 1), lambda i: (i, 0)),
            pl.BlockSpec((3 * EMBED_SIZE, WORD_DIM), lambda i: (0, 0)),
            pl.BlockSpec((3 * EMBED_SIZE, EMBED_SIZE), lambda i: (0, 0)),
            pl.BlockSpec((1, 3 * EMBED_SIZE), lambda i: (0, 0)),
            pl.BlockSpec((1, 3 * EMBED_SIZE), lambda i: (0, 0)),
        ],
        out_specs=[
            pl.BlockSpec((_BB, EMBED_SIZE), lambda i: (i, 0)),
            pl.BlockSpec((_BB, SEQ, WORD_DIM), lambda i: (i, 0, 0)),
        ],
        out_shape=[
            jax.ShapeDtypeStruct((BATCH, EMBED_SIZE), jnp.float32),
            jax.ShapeDtypeStruct((BATCH, SEQ, WORD_DIM), jnp.float32),
        ],
    )(cap_tm, lengths2d, W_ih, W_hh, b_ih2d, b_hh2d)


def kernel(x, lengths, embed_table, W_ih, W_hh, b_ih, b_hh):
    idx_tm = x.T.reshape(-1).astype(jnp.int32)
    cap_flat = _sc_gather(idx_tm, embed_table)
    cap_tm = cap_flat.reshape(SEQ, BATCH, WORD_DIM)
    out, cap_emb = _tc_gru(
        cap_tm,
        lengths.reshape(BATCH, 1).astype(jnp.int32),
        W_ih,
        W_hh,
        b_ih.reshape(1, -1),
        b_hh.reshape(1, -1),
    )
    return (out, cap_emb)


# speedup vs baseline: 2.3669x; 1.0041x over previous
import functools

import jax
import jax.numpy as jnp
from jax import lax
from jax.experimental import pallas as pl
from jax.experimental.pallas import tpu as pltpu
from jax.experimental.pallas import tpu_sc as plsc

VOCAB = 100000
WORD_DIM = 128
EMBED_SIZE = 512
BATCH = 1024
SEQ = 20


_N_ROWS = BATCH * SEQ
_IDX_CHUNK = 128


def _sc_gather(idx_flat, table):
    info = plsc.get_sparse_core_info()
    nc, ns = info.num_cores, info.num_subcores
    nw = nc * ns
    rows_per_w = _N_ROWS // nw
    assert rows_per_w % _IDX_CHUNK == 0
    n_chunks = rows_per_w // _IDX_CHUNK

    mesh = plsc.VectorSubcoreMesh(core_axis_name="c", subcore_axis_name="s")

    @functools.partial(
        pl.kernel,
        mesh=mesh,
        out_type=jax.ShapeDtypeStruct((_N_ROWS, WORD_DIM), jnp.float32),
        scratch_types=[
            pltpu.VMEM((rows_per_w,), jnp.int32),
            pltpu.VMEM((rows_per_w, WORD_DIM), jnp.float32),
            pltpu.SemaphoreType.DMA,
        ],
    )
    def gather_k(idx_hbm, table_hbm, out_hbm, idx_v, rows_v, sem):
        wid = lax.axis_index("s") * nc + lax.axis_index("c")
        base = wid * rows_per_w
        pltpu.sync_copy(idx_hbm.at[pl.ds(base, rows_per_w)], idx_v)
        copies = []
        for j in range(n_chunks):
            sl = pl.ds(j * _IDX_CHUNK, _IDX_CHUNK)
            copies.append(
                pltpu.async_copy(table_hbm.at[idx_v.at[sl]], rows_v.at[sl], sem)
            )
        for c in copies:
            c.wait()
        pltpu.sync_copy(rows_v, out_hbm.at[pl.ds(base, rows_per_w)])

    return gather_k(idx_flat, table)


_BB = 512


_NSUB = 2


def _gru_block(cap_ref, len_ref, wih_ref, whh_ref, bih_ref, bhh_ref,
               out_ref, cap3_ref):
    wih_b = wih_ref[...].astype(jnp.bfloat16)
    whh_b = whh_ref[...].astype(jnp.bfloat16)
    bih = bih_ref[...]
    bhh = bhh_ref[...]

    dn = (((1,), (1,)), ((), ()))
    H = EMBED_SIZE
    SB = _BB // _NSUB
    bsum = bih + bhh
    neg = jnp.finfo(jnp.float32).min
    h = [jnp.zeros((SB, H), jnp.float32) for _ in range(_NSUB)]
    acc = [jnp.full((SB, H), neg, jnp.float32) for _ in range(_NSUB)]
    lens = [len_ref[...][s * SB:(s + 1) * SB, :] for s in range(_NSUB)]

    for t in range(SEQ):
        xt = cap_ref[t]
        cap3_ref[:, t, :] = xt
        for s in range(_NSUB):
            xs = xt[s * SB:(s + 1) * SB, :].astype(jnp.bfloat16)
            gi = lax.dot_general(xs, wih_b, dn,
                                 preferred_element_type=jnp.float32)
            gh = lax.dot_general(h[s].astype(jnp.bfloat16), whh_b, dn,
                                 preferred_element_type=jnp.float32)
            rz = gi[:, :2 * H] + gh[:, :2 * H] + bsum[:, :2 * H]
            rz = 0.5 * jnp.tanh(0.5 * rz) + 0.5
            r = rz[:, :H]
            z = rz[:, H:]
            hn = gh[:, 2 * H:] + bhh[:, 2 * H:]
            n = jnp.tanh(gi[:, 2 * H:] + bih[:, 2 * H:] + r * hn)
            h[s] = n + z * (h[s] - n)
            valid = t < lens[s]
            acc[s] = jnp.where(valid, jnp.maximum(acc[s], h[s]), acc[s])

    out_ref[...] = jnp.concatenate(acc, axis=0)


def _tc_gru(cap_tm, lengths2d, W_ih, W_hh, b_ih2d, b_hh2d):
    grid = BATCH // _BB
    return pl.pallas_call(
        _gru_block,
        grid=(grid,),
        in_specs=[
            pl.BlockSpec((SEQ, _BB, WORD_DIM), lambda i: (0, i, 0)),
            pl.BlockSpec((_BB, 1), lambda i: (i, 0)),
            pl.BlockSpec((3 * EMBED_SIZE, WORD_DIM), lambda i: (0, 0)),
            pl.BlockSpec((3 * EMBED_SIZE, EMBED_SIZE), lambda i: (0, 0)),
            pl.BlockSpec((1, 3 * EMBED_SIZE), lambda i: (0, 0)),
            pl.BlockSpec((1, 3 * EMBED_SIZE), lambda i: (0, 0)),
        ],
        out_specs=[
            pl.BlockSpec((_BB, EMBED_SIZE), lambda i: (i, 0)),
            pl.BlockSpec((_BB, SEQ, WORD_DIM), lambda i: (i, 0, 0)),
        ],
        out_shape=[
            jax.ShapeDtypeStruct((BATCH, EMBED_SIZE), jnp.float32),
            jax.ShapeDtypeStruct((BATCH, SEQ, WORD_DIM), jnp.float32),
        ],
    )(cap_tm, lengths2d, W_ih, W_hh, b_ih2d, b_hh2d)


def kernel(x, lengths, embed_table, W_ih, W_hh, b_ih, b_hh):
    idx_tm = x.T.reshape(-1).astype(jnp.int32)
    cap_flat = _sc_gather(idx_tm, embed_table)
    cap_tm = cap_flat.reshape(SEQ, BATCH, WORD_DIM)
    out, cap_emb = _tc_gru(
        cap_tm,
        lengths.reshape(BATCH, 1).astype(jnp.int32),
        W_ih,
        W_hh,
        b_ih.reshape(1, -1),
        b_hh.reshape(1, -1),
    )
    return (out, cap_emb)
